# baseline (device time: 354069 ns/iter reference)
import jax
import jax.numpy as jnp
from jax import lax
from jax.experimental import pallas as pl
from jax.experimental.pallas import tpu as pltpu

N_DEV = 16
M_PER = 256
N_COLS = 8192
HALF = N_COLS // 2
LANES = 4
LW = HALF // LANES
COMM_DTYPE = jnp.bfloat16

RING = [0, 4, 8, 12, 15, 11, 7, 3, 2, 6, 10, 14, 13, 9, 5, 1]
INV = [RING.index(d) for d in range(N_DEV)]
RIGHT = [RING[(INV[d] + 1) % N_DEV] for d in range(N_DEV)]
LEFT = [RING[(INV[d] - 1) % N_DEV] for d in range(N_DEV)]
CHUNK = [
    [[RING[(INV[d] - s - 1) % N_DEV] for d in range(N_DEV)]
     for s in range(N_DEV)],
    [[RING[(INV[d] + s + 1) % N_DEV] for d in range(N_DEV)]
     for s in range(N_DEV)],
]


def _lookup(table, idx):
    v = jnp.int32(table[0])
    for j in range(1, N_DEV):
        v = jnp.where(idx == j, jnp.int32(table[j]), v)
    return v


def kernel(x, w_mat):
    n_dl = 2 * LANES

    def body(x_ref, w_ref, out_ref, xbf_ref, wbf_ref, *rest):
        recv = [rest[dir * LANES:(dir + 1) * LANES] for dir in range(2)]
        send = [rest[n_dl + dir * LANES:n_dl + (dir + 1) * LANES]
                for dir in range(2)]
        send_sem = [rest[2 * n_dl + dir * LANES:2 * n_dl + (dir + 1) * LANES]
                    for dir in range(2)]
        recv_sem = [rest[3 * n_dl + dir * LANES:3 * n_dl + (dir + 1) * LANES]
                    for dir in range(2)]
        cred = [rest[4 * n_dl + dir * LANES:4 * n_dl + (dir + 1) * LANES]
                for dir in range(2)]

        d = lax.axis_index("i")
        my_r = _lookup(RIGHT, d)
        my_l = _lookup(LEFT, d)
        dst = [my_r, my_l]
        src = [my_l, my_r]

        xbf_ref[:, :] = x_ref[:, :].astype(COMM_DTYPE)
        wbf_ref[:, :] = w_ref[:, :].astype(COMM_DTYPE)

        for dir in range(2):
            for ln in range(LANES):
                pl.semaphore_signal(cred[dir][ln], inc=2,
                                    device_id=(src[dir],),
                                    device_id_type=pl.DeviceIdType.MESH)

        barrier = pltpu.get_barrier_semaphore()
        for nbr in (my_l, my_r):
            pl.semaphore_signal(barrier, inc=1, device_id=(nbr,),
                                device_id_type=pl.DeviceIdType.MESH)
        pl.semaphore_wait(barrier, 2)

        dims = (((1,), (0,)), ((), ()))
        pend = [[[None, None] for _ in range(LANES)] for _ in range(2)]

        for s in range(N_DEV):
            slot = s % 2
            nslot = (s + 1) % 2
            p = []
            for dir in range(2):
                c = _lookup(CHUNK[dir][s], d)
                p.append(lax.dot_general(
                    xbf_ref[pl.ds(c * M_PER, M_PER), :],
                    wbf_ref[:, dir * HALF:(dir + 1) * HALF],
                    dims, preferred_element_type=jnp.float32))

            for ln in range(LANES):
                for dir in range(2):
                    pslice = p[dir][:, ln * LW:(ln + 1) * LW]
                    if s == 0:
                        vv = pslice
                    else:
                        rd = pltpu.make_async_remote_copy(
                            src_ref=send[dir][ln].at[slot],
                            dst_ref=recv[dir][ln].at[slot],
                            send_sem=send_sem[dir][ln].at[slot],
                            recv_sem=recv_sem[dir][ln].at[slot],
                            device_id=(src[dir],),
                            device_id_type=pl.DeviceIdType.MESH)
                        rd.wait_recv()
                        vv = recv[dir][ln][slot].astype(jnp.float32) + pslice
                        if s <= N_DEV - 3:
                            pl.semaphore_signal(
                                cred[dir][ln], inc=1, device_id=(src[dir],),
                                device_id_type=pl.DeviceIdType.MESH)

                    if s < N_DEV - 1:
                        if pend[dir][ln][slot] is not None:
                            pend[dir][ln][slot].wait_send()
                        send[dir][ln][slot, :, :] = vv.astype(COMM_DTYPE)
                        pl.semaphore_wait(cred[dir][ln], 1)
                        sd = pltpu.make_async_remote_copy(
                            src_ref=send[dir][ln].at[slot],
                            dst_ref=recv[dir][ln].at[nslot],
                            send_sem=send_sem[dir][ln].at[slot],
                            recv_sem=recv_sem[dir][ln].at[nslot],
                            device_id=(dst[dir],),
                            device_id_type=pl.DeviceIdType.MESH)
                        sd.start()
                        pend[dir][ln][slot] = sd
                    else:
                        col = dir * HALF + ln * LW
                        out_ref[:, col:col + LW] = jnp.maximum(vv, 0.0)

        for dir in range(2):
            for ln in range(LANES):
                for sl in range(2):
                    if pend[dir][ln][sl] is not None:
                        pend[dir][ln][sl].wait_send()

    comm_bufs = [pltpu.VMEM((2, M_PER, LW), COMM_DTYPE)
                 for _ in range(2 * n_dl)]
    dma_sems = [pltpu.SemaphoreType.DMA((2,))
                for _ in range(2 * n_dl)]
    cred_sems = [pltpu.SemaphoreType.REGULAR for _ in range(n_dl)]

    return pl.pallas_call(
        body,
        out_shape=jax.ShapeDtypeStruct((M_PER, N_COLS), jnp.float32),
        in_specs=[pl.BlockSpec(memory_space=pltpu.VMEM),
                  pl.BlockSpec(memory_space=pltpu.VMEM)],
        out_specs=pl.BlockSpec(memory_space=pltpu.VMEM),
        scratch_shapes=(
            [pltpu.VMEM(x.shape, COMM_DTYPE),
             pltpu.VMEM(w_mat.shape, COMM_DTYPE)]
            + comm_bufs + dma_sems + cred_sems
        ),
        compiler_params=pltpu.CompilerParams(collective_id=0),
    )(x, w_mat)


# device time: 352676 ns/iter; 1.0039x vs baseline; 1.0039x over previous
import jax
import jax.numpy as jnp
from jax import lax
from jax.experimental import pallas as pl
from jax.experimental.pallas import tpu as pltpu

N_DEV = 16
M_PER = 256
N_COLS = 8192
HALF = N_COLS // 2
LANES = 2
LW = HALF // LANES
COMM_DTYPE = jnp.bfloat16

RING = [0, 4, 8, 12, 15, 11, 7, 3, 2, 6, 10, 14, 13, 9, 5, 1]
INV = [RING.index(d) for d in range(N_DEV)]
RIGHT = [RING[(INV[d] + 1) % N_DEV] for d in range(N_DEV)]
LEFT = [RING[(INV[d] - 1) % N_DEV] for d in range(N_DEV)]
CHUNK = [
    [[RING[(INV[d] - s - 1) % N_DEV] for d in range(N_DEV)]
     for s in range(N_DEV)],
    [[RING[(INV[d] + s + 1) % N_DEV] for d in range(N_DEV)]
     for s in range(N_DEV)],
]


def _lookup(table, idx):
    v = jnp.int32(table[0])
    for j in range(1, N_DEV):
        v = jnp.where(idx == j, jnp.int32(table[j]), v)
    return v


def kernel(x, w_mat):
    n_dl = 2 * LANES

    def body(x_ref, w_ref, out_ref, *rest):
        recv = [rest[dir * LANES:(dir + 1) * LANES] for dir in range(2)]
        send = [rest[n_dl + dir * LANES:n_dl + (dir + 1) * LANES]
                for dir in range(2)]
        send_sem = [rest[2 * n_dl + dir * LANES:2 * n_dl + (dir + 1) * LANES]
                    for dir in range(2)]
        recv_sem = [rest[3 * n_dl + dir * LANES:3 * n_dl + (dir + 1) * LANES]
                    for dir in range(2)]
        cred = [rest[4 * n_dl + dir * LANES:4 * n_dl + (dir + 1) * LANES]
                for dir in range(2)]

        d = lax.axis_index("i")
        my_r = _lookup(RIGHT, d)
        my_l = _lookup(LEFT, d)
        dst = [my_r, my_l]
        src = [my_l, my_r]

        for dir in range(2):
            for ln in range(LANES):
                pl.semaphore_signal(cred[dir][ln], inc=2,
                                    device_id=(src[dir],),
                                    device_id_type=pl.DeviceIdType.MESH)

        barrier = pltpu.get_barrier_semaphore()
        for nbr in (my_l, my_r):
            pl.semaphore_signal(barrier, inc=1, device_id=(nbr,),
                                device_id_type=pl.DeviceIdType.MESH)
        pl.semaphore_wait(barrier, 2)

        dims = (((1,), (0,)), ((), ()))
        pend = [[[None, None] for _ in range(LANES)] for _ in range(2)]

        for s in range(N_DEV):
            slot = s % 2
            nslot = (s + 1) % 2
            xs = []
            for dir in range(2):
                c = _lookup(CHUNK[dir][s], d)
                xs.append(
                    x_ref[pl.ds(c * M_PER, M_PER), :].astype(COMM_DTYPE))

            for ln in range(LANES):
                for dir in range(2):
                    col = dir * HALF + ln * LW
                    pslice = lax.dot_general(
                        xs[dir], w_ref[:, col:col + LW].astype(COMM_DTYPE),
                        dims, preferred_element_type=jnp.float32)
                    if s == 0:
                        vv = pslice
                    else:
                        rd = pltpu.make_async_remote_copy(
                            src_ref=send[dir][ln].at[slot],
                            dst_ref=recv[dir][ln].at[slot],
                            send_sem=send_sem[dir][ln].at[slot],
                            recv_sem=recv_sem[dir][ln].at[slot],
                            device_id=(src[dir],),
                            device_id_type=pl.DeviceIdType.MESH)
                        rd.wait_recv()
                        vv = recv[dir][ln][slot].astype(jnp.float32) + pslice
                        if s <= N_DEV - 3:
                            pl.semaphore_signal(
                                cred[dir][ln], inc=1, device_id=(src[dir],),
                                device_id_type=pl.DeviceIdType.MESH)

                    if s < N_DEV - 1:
                        if pend[dir][ln][slot] is not None:
                            pend[dir][ln][slot].wait_send()
                        send[dir][ln][slot, :, :] = vv.astype(COMM_DTYPE)
                        pl.semaphore_wait(cred[dir][ln], 1)
                        sd = pltpu.make_async_remote_copy(
                            src_ref=send[dir][ln].at[slot],
                            dst_ref=recv[dir][ln].at[nslot],
                            send_sem=send_sem[dir][ln].at[slot],
                            recv_sem=recv_sem[dir][ln].at[nslot],
                            device_id=(dst[dir],),
                            device_id_type=pl.DeviceIdType.MESH)
                        sd.start()
                        pend[dir][ln][slot] = sd
                    else:
                        out_ref[:, col:col + LW] = jnp.maximum(vv, 0.0)

        for dir in range(2):
            for ln in range(LANES):
                for sl in range(2):
                    if pend[dir][ln][sl] is not None:
                        pend[dir][ln][sl].wait_send()

    comm_bufs = [pltpu.VMEM((2, M_PER, LW), COMM_DTYPE)
                 for _ in range(2 * n_dl)]
    dma_sems = [pltpu.SemaphoreType.DMA((2,))
                for _ in range(2 * n_dl)]
    cred_sems = [pltpu.SemaphoreType.REGULAR for _ in range(n_dl)]

    return pl.pallas_call(
        body,
        out_shape=jax.ShapeDtypeStruct((M_PER, N_COLS), jnp.float32),
        in_specs=[pl.BlockSpec(memory_space=pltpu.VMEM),
                  pl.BlockSpec(memory_space=pltpu.VMEM)],
        out_specs=pl.BlockSpec(memory_space=pltpu.VMEM),
        scratch_shapes=(comm_bufs + dma_sems + cred_sems),
        compiler_params=pltpu.CompilerParams(collective_id=0),
    )(x, w_mat)
